# baseline (speedup 1.0000x reference)
import jax
import jax.numpy as jnp
from jax import lax
from jax.experimental import pallas as pl
from jax.experimental.pallas import tpu as pltpu
from jax.experimental.pallas import tpu_sc as plsc

N = 10000
D = 128
DH = D // 2
E = 320000
NP = 10240
NC = 2
NS = 16
NW = NC * NS
CH = 125
CPW = E // NS // CH
RT = NP // NS
SB = 40

_mesh = plsc.VectorSubcoreMesh(core_axis_name="c", subcore_axis_name="s")

_f32 = jnp.float32


def _pipeline_phase(src_ref, gidx_hbm, sidx_hbm, gbase, sbase, acc, cnt,
                    gab, eab, rows, onesv, semi, semg, semo, sems,
                    cnt_parity=None):
    def idx_load(j):
        slot = lax.rem(j, 6)
        pltpu.async_copy(gidx_hbm.at[gbase + j], gab.at[slot], semi)
        pltpu.async_copy(sidx_hbm.at[sbase + j], eab.at[slot], semi)

    def idx_wait():
        pltpu.make_async_copy(gidx_hbm.at[0], gab.at[0], semi).wait()
        pltpu.make_async_copy(sidx_hbm.at[0], eab.at[0], semi).wait()

    def gather(j):
        pltpu.async_copy(src_ref.at[gab.at[lax.rem(j, 6)]],
                         rows.at[lax.rem(j, 4)], semg)

    def ones_drain():
        pltpu.make_async_copy(onesv, cnt.at[eab.at[0]], semo).wait()

    idx_load(0)
    idx_load(1)
    idx_load(2)
    idx_wait()
    gather(0)
    idx_wait()
    gather(1)

    def body(j, carry):
        @pl.when(j >= 2)
        def _():
            pltpu.make_async_copy(rows.at[0], acc.at[eab.at[0]], sems).wait()

        if cnt_parity is None:
            @pl.when(j >= 2)
            def _():
                ones_drain()
        else:
            @pl.when((j >= 2) & (lax.rem(j, 2) == cnt_parity))
            def _():
                ones_drain()

        @pl.when(j + 3 < CPW)
        def _():
            idx_load(j + 3)

        @pl.when(j + 2 < CPW)
        def _():
            idx_wait()
            gather(j + 2)

        pltpu.make_async_copy(src_ref.at[pl.ds(0, CH)], rows.at[0],
                              semg).wait()
        slot = lax.rem(j, 6)
        pltpu.async_copy(rows.at[lax.rem(j, 4)], acc.at[eab.at[slot]],
                         sems, add=True)
        if cnt_parity is None:
            pltpu.async_copy(onesv, cnt.at[eab.at[slot]], semo, add=True)
        else:
            @pl.when(lax.rem(j, 2) == cnt_parity)
            def _():
                pltpu.async_copy(onesv, cnt.at[eab.at[slot]], semo,
                                 add=True)
        return carry

    lax.fori_loop(0, CPW, body, 0)
    pltpu.make_async_copy(rows.at[0], acc.at[eab.at[0]], sems).wait()
    pltpu.make_async_copy(rows.at[0], acc.at[eab.at[0]], sems).wait()
    ones_drain()
    if cnt_parity is None:
        ones_drain()


def _sc_body(xs_hbm, n3_hbm, e3_hbm, zrow_hbm, zcnt_hbm,
             ones_hbm,
             o_out, dn_out, e_out,
             gab, eab, rows, onesv, av, cv, accA, accB, cnt,
             semi, semg, semo, sems):
    cidx = lax.axis_index("c")
    sidx = lax.axis_index("s")
    w = cidx * NS + sidx
    pltpu.sync_copy(zrow_hbm, accA.at[pl.ds(sidx * RT, RT)])
    pltpu.sync_copy(zrow_hbm, accB.at[pl.ds(sidx * RT, RT)])
    pltpu.sync_copy(zcnt_hbm, cnt.at[pl.ds(sidx * RT, RT)])
    pltpu.sync_copy(ones_hbm, onesv)
    plsc.subcore_barrier()

    _pipeline_phase(xs_hbm.at[pl.ds(cidx * N, N)], n3_hbm, e3_hbm,
                    sidx * CPW, sidx * CPW, accA, cnt,
                    gab, eab, rows, onesv, semi, semg, semo, sems)
    plsc.subcore_barrier()

    def blk_s(p, carry):
        base_r = sidx * RT + p * SB
        pltpu.sync_copy(accA.at[pl.ds(base_r, SB)], av)
        pltpu.sync_copy(cnt.at[pl.ds(base_r, SB)], cv)

        def body_s(r, c2):
            c16 = cv[r]
            c16 = jnp.where(c16 == 0.0, 1.0, c16)
            for k in range(DH // 16):
                av[r, pl.ds(k * 16, 16)] = av[r, pl.ds(k * 16, 16)] / c16
            return c2

        lax.fori_loop(0, SB, body_s, 0)
        pltpu.sync_copy(av, e_out.at[pl.ds(cidx * NP + base_r, SB)])
        return carry

    lax.fori_loop(0, RT // SB, blk_s, 0)
    pltpu.sync_copy(zcnt_hbm, cnt.at[pl.ds(sidx * RT, RT)])
    plsc.subcore_barrier()

    _pipeline_phase(e_out.at[pl.ds(cidx * NP, NP)], e3_hbm, n3_hbm,
                    sidx * CPW, sidx * CPW, accB, cnt,
                    gab, eab, rows, onesv, semi, semg, semo, sems,
                    cnt_parity=cidx)
    plsc.subcore_barrier()

    base = cidx * NP + sidx * RT
    pltpu.sync_copy(accB.at[pl.ds(sidx * RT, RT)], o_out.at[pl.ds(base, RT)])
    pltpu.sync_copy(cnt.at[pl.ds(sidx * RT, RT)], dn_out.at[pl.ds(base, RT)])


_sc_seg = pl.kernel(
    _sc_body,
    out_type=[
        jax.ShapeDtypeStruct((NC * NP, DH), _f32),
        jax.ShapeDtypeStruct((NC * NP, 16), _f32),
        jax.ShapeDtypeStruct((NC * NP, DH), _f32),
    ],
    mesh=_mesh,
    scratch_types=[
        pltpu.VMEM((6, CH), jnp.int32),
        pltpu.VMEM((6, CH), jnp.int32),
        pltpu.VMEM((4, CH, DH), _f32),
        pltpu.VMEM((CH, 16), _f32),
        pltpu.VMEM((SB, DH), _f32),
        pltpu.VMEM((SB, 16), _f32),
        pltpu.VMEM_SHARED((NP, DH), _f32),
        pltpu.VMEM_SHARED((NP, DH), _f32),
        pltpu.VMEM_SHARED((NP, 16), _f32),
        pltpu.SemaphoreType.DMA,
        pltpu.SemaphoreType.DMA,
        pltpu.SemaphoreType.DMA,
        pltpu.SemaphoreType.DMA,
    ],
    compiler_params=pltpu.CompilerParams(use_tc_tiling_on_sc=False),
)


_RB = 2000


def _tc_body(o_ref, dn_ref, th_ref, b_ref, out_ref):
    s2 = jnp.concatenate([o_ref[0], o_ref[1]], axis=1)
    dn = dn_ref[0][:, 0:1] + dn_ref[1][:, 0:1]
    dn = jnp.where(dn == 0.0, 1.0, dn)
    out_ref[...] = jnp.dot(s2, th_ref[...],
                           preferred_element_type=_f32) / dn + b_ref[...]


def _tc_fin(o_part, dn_part, theta, bias2d):
    return pl.pallas_call(
        _tc_body,
        grid=(N // _RB,),
        in_specs=[
            pl.BlockSpec((NC, _RB, DH), lambda i: (0, i, 0)),
            pl.BlockSpec((NC, _RB, 16), lambda i: (0, i, 0)),
            pl.BlockSpec((D, D), lambda i: (0, 0)),
            pl.BlockSpec((1, D), lambda i: (0, 0)),
        ],
        out_specs=pl.BlockSpec((_RB, D), lambda i: (i, 0)),
        out_shape=jax.ShapeDtypeStruct((N, D), _f32),
    )(o_part, dn_part, theta, bias2d)


def kernel(x, hyper_edge_index, theta, bias):
    idx = hyper_edge_index.astype(jnp.int32)
    n3 = idx[0].reshape(NS * CPW, CH)
    e3 = idx[1].reshape(NS * CPW, CH)
    xs = jnp.concatenate([x[:, :DH], x[:, DH:]], 0)
    zrow = jnp.zeros((RT, DH), _f32)
    zcnt = jnp.zeros((RT, 16), _f32)
    ones = jnp.ones((CH, 16), _f32)
    o_part, dn_part, _ = _sc_seg(xs, n3, e3, zrow, zcnt, ones)
    return _tc_fin(o_part.reshape(NC, NP, DH), dn_part.reshape(NC, NP, 16),
                   theta, bias.reshape(1, D))


# speedup vs baseline: 1.0160x; 1.0160x over previous
import jax
import jax.numpy as jnp
from jax import lax
from jax.experimental import pallas as pl
from jax.experimental.pallas import tpu as pltpu
from jax.experimental.pallas import tpu_sc as plsc

N = 10000
D = 128
DH = D // 2
E = 320000
NP = 10240
NC = 2
NS = 16
NW = NC * NS
CH = 125
CPW = E // NS // CH
RT = NP // NS
SB = 80

_mesh = plsc.VectorSubcoreMesh(core_axis_name="c", subcore_axis_name="s")

_f32 = jnp.float32


def _pipeline_phase(src_ref, gidx_hbm, sidx_hbm, gbase, sbase, acc, cnt,
                    gab, eab, rows, onesv, semi, semg, semo, sems,
                    cnt_parity=None):
    def idx_load(j):
        slot = lax.rem(j, 6)
        pltpu.async_copy(gidx_hbm.at[gbase + j], gab.at[slot], semi)
        pltpu.async_copy(sidx_hbm.at[sbase + j], eab.at[slot], semi)

    def idx_wait():
        pltpu.make_async_copy(gidx_hbm.at[0], gab.at[0], semi).wait()
        pltpu.make_async_copy(sidx_hbm.at[0], eab.at[0], semi).wait()

    def gather(j):
        pltpu.async_copy(src_ref.at[gab.at[lax.rem(j, 6)]],
                         rows.at[lax.rem(j, 4)], semg)

    def ones_drain():
        pltpu.make_async_copy(onesv, cnt.at[eab.at[0]], semo).wait()

    idx_load(0)
    idx_load(1)
    idx_load(2)
    idx_wait()
    gather(0)
    idx_wait()
    gather(1)

    def body(j, carry):
        @pl.when(j >= 2)
        def _():
            pltpu.make_async_copy(rows.at[0], acc.at[eab.at[0]], sems).wait()

        if cnt_parity is None:
            @pl.when(j >= 2)
            def _():
                ones_drain()
        else:
            @pl.when((j >= 2) & (lax.rem(j, 2) == cnt_parity))
            def _():
                ones_drain()

        @pl.when(j + 3 < CPW)
        def _():
            idx_load(j + 3)

        @pl.when(j + 2 < CPW)
        def _():
            idx_wait()
            gather(j + 2)

        pltpu.make_async_copy(src_ref.at[pl.ds(0, CH)], rows.at[0],
                              semg).wait()
        slot = lax.rem(j, 6)
        pltpu.async_copy(rows.at[lax.rem(j, 4)], acc.at[eab.at[slot]],
                         sems, add=True)
        if cnt_parity is None:
            pltpu.async_copy(onesv, cnt.at[eab.at[slot]], semo, add=True)
        else:
            @pl.when(lax.rem(j, 2) == cnt_parity)
            def _():
                pltpu.async_copy(onesv, cnt.at[eab.at[slot]], semo,
                                 add=True)
        return carry

    lax.fori_loop(0, CPW, body, 0)
    pltpu.make_async_copy(rows.at[0], acc.at[eab.at[0]], sems).wait()
    pltpu.make_async_copy(rows.at[0], acc.at[eab.at[0]], sems).wait()
    ones_drain()
    if cnt_parity is None:
        ones_drain()


def _sc_body(xs_hbm, ga_hbm, e3_hbm, eb_hbm, n3_hbm, zrow_hbm, zcnt_hbm,
             ones_hbm,
             o_out, dn_out, e_out,
             gab, eab, rows, onesv, cv, accA, accB, cnt,
             semi, semg, semo, sems):
    cidx = lax.axis_index("c")
    sidx = lax.axis_index("s")
    w = cidx * NS + sidx
    pltpu.sync_copy(zrow_hbm, accA.at[pl.ds(sidx * RT, RT)])
    pltpu.sync_copy(zrow_hbm, accB.at[pl.ds(sidx * RT, RT)])
    pltpu.sync_copy(zcnt_hbm, cnt.at[pl.ds(sidx * RT, RT)])
    pltpu.sync_copy(ones_hbm, onesv)
    plsc.subcore_barrier()

    _pipeline_phase(xs_hbm, ga_hbm, e3_hbm, w * CPW, sidx * CPW, accA, cnt,
                    gab, eab, rows, onesv, semi, semg, semo, sems)
    plsc.subcore_barrier()

    NBLK = RT // SB

    def s_load(p):
        base_r = sidx * RT + p * SB
        pltpu.async_copy(accA.at[pl.ds(base_r, SB)],
                         rows.at[lax.rem(p, 2), pl.ds(0, SB)], semg)
        pltpu.async_copy(cnt.at[pl.ds(base_r, SB)],
                         cv.at[lax.rem(p, 2)], semi)

    def s_wait():
        pltpu.make_async_copy(accA.at[pl.ds(0, SB)],
                              rows.at[0, pl.ds(0, SB)], semg).wait()
        pltpu.make_async_copy(cnt.at[pl.ds(0, SB)], cv.at[0], semi).wait()

    def s_drain_store():
        pltpu.make_async_copy(rows.at[0, pl.ds(0, SB)],
                              e_out.at[pl.ds(0, SB)], sems).wait()

    s_load(0)

    def blk_s(p, carry):
        @pl.when(p >= 1)
        def _():
            s_drain_store()

        @pl.when(p + 1 < NBLK)
        def _():
            s_load(p + 1)

        s_wait()
        slot = lax.rem(p, 2)

        def body_s(r, c2):
            c16 = cv[slot, r]
            c16 = jnp.where(c16 == 0.0, 1.0, c16)
            for k in range(DH // 16):
                rows[slot, r, pl.ds(k * 16, 16)] = (
                    rows[slot, r, pl.ds(k * 16, 16)] / c16)
            return c2

        lax.fori_loop(0, SB, body_s, 0)
        base_r = sidx * RT + p * SB
        pltpu.async_copy(rows.at[slot, pl.ds(0, SB)],
                         e_out.at[pl.ds(cidx * NP + base_r, SB)], sems)
        return carry

    lax.fori_loop(0, NBLK, blk_s, 0)
    s_drain_store()
    pltpu.sync_copy(zcnt_hbm, cnt.at[pl.ds(sidx * RT, RT)])
    plsc.subcore_barrier()

    _pipeline_phase(e_out, eb_hbm, n3_hbm, w * CPW, sidx * CPW, accB, cnt,
                    gab, eab, rows, onesv, semi, semg, semo, sems,
                    cnt_parity=cidx)
    plsc.subcore_barrier()

    base = cidx * NP + sidx * RT
    pltpu.sync_copy(accB.at[pl.ds(sidx * RT, RT)], o_out.at[pl.ds(base, RT)])
    pltpu.sync_copy(cnt.at[pl.ds(sidx * RT, RT)], dn_out.at[pl.ds(base, RT)])


_sc_seg = pl.kernel(
    _sc_body,
    out_type=[
        jax.ShapeDtypeStruct((NC * NP, DH), _f32),
        jax.ShapeDtypeStruct((NC * NP, 16), _f32),
        jax.ShapeDtypeStruct((NC * NP, DH), _f32),
    ],
    mesh=_mesh,
    scratch_types=[
        pltpu.VMEM((6, CH), jnp.int32),
        pltpu.VMEM((6, CH), jnp.int32),
        pltpu.VMEM((4, CH, DH), _f32),
        pltpu.VMEM((CH, 16), _f32),
        pltpu.VMEM((2, SB, 16), _f32),
        pltpu.VMEM_SHARED((NP, DH), _f32),
        pltpu.VMEM_SHARED((NP, DH), _f32),
        pltpu.VMEM_SHARED((NP, 16), _f32),
        pltpu.SemaphoreType.DMA,
        pltpu.SemaphoreType.DMA,
        pltpu.SemaphoreType.DMA,
        pltpu.SemaphoreType.DMA,
    ],
    compiler_params=pltpu.CompilerParams(use_tc_tiling_on_sc=False),
)


_RB = 2000


def _tc_body(o_ref, dn_ref, th_ref, b_ref, out_ref):
    s2 = jnp.concatenate([o_ref[0], o_ref[1]], axis=1)
    dn = dn_ref[0][:, 0:1] + dn_ref[1][:, 0:1]
    dn = jnp.where(dn == 0.0, 1.0, dn)
    out_ref[...] = jnp.dot(s2, th_ref[...],
                           preferred_element_type=_f32) / dn + b_ref[...]


def _tc_fin(o_part, dn_part, theta, bias2d):
    return pl.pallas_call(
        _tc_body,
        grid=(N // _RB,),
        in_specs=[
            pl.BlockSpec((NC, _RB, DH), lambda i: (0, i, 0)),
            pl.BlockSpec((NC, _RB, 16), lambda i: (0, i, 0)),
            pl.BlockSpec((D, D), lambda i: (0, 0)),
            pl.BlockSpec((1, D), lambda i: (0, 0)),
        ],
        out_specs=pl.BlockSpec((_RB, D), lambda i: (i, 0)),
        out_shape=jax.ShapeDtypeStruct((N, D), _f32),
    )(o_part, dn_part, theta, bias2d)


def kernel(x, hyper_edge_index, theta, bias):
    idx = hyper_edge_index.astype(jnp.int32)
    n3 = idx[0].reshape(NS * CPW, CH)
    e3 = idx[1].reshape(NS * CPW, CH)
    ga = jnp.concatenate([n3, n3 + N], 0)
    eb = jnp.concatenate([e3, e3 + NP], 0)
    xs = jnp.concatenate([x[:, :DH], x[:, DH:]], 0)
    zrow = jnp.zeros((RT, DH), _f32)
    zcnt = jnp.zeros((RT, 16), _f32)
    ones = jnp.ones((CH, 16), _f32)
    o_part, dn_part, _ = _sc_seg(xs, ga, e3, eb, n3, zrow, zcnt, ones)
    return _tc_fin(o_part.reshape(NC, NP, DH), dn_part.reshape(NC, NP, 16),
                   theta, bias.reshape(1, D))
